# baseline (speedup 1.0000x reference)
import functools

import jax
import jax.numpy as jnp
from jax import lax
from jax.experimental import pallas as pl
from jax.experimental.pallas import tpu as pltpu
from jax.experimental.pallas import tpu_sc as plsc

N = 10000
E = 320000
IN_C = 128
HID = 128
OUT_C = 16

NC = 2
NS = 16
NW = NC * NS
C = 128
EPT = 10240
E_PAD = EPT * NW
NCHUNK = EPT // C
RPT = 624

_f32 = jnp.float32
_i32 = jnp.int32

_MESH = plsc.VectorSubcoreMesh(
    core_axis_name="c", subcore_axis_name="s", num_cores=NC, num_subcores=NS)

_CP = pltpu.CompilerParams(needs_layout_passes=False, use_tc_tiling_on_sc=False)


def _splat(vec_ref, e):
  return plsc.load_gather(vec_ref, [jnp.full((16,), e, _i32)])


def _deint(ebuf, cidx, ridx, aval):
  for k in range(C // 16):
    ro = lax.iota(_i32, 16) + (16 * k)
    ridx[pl.ds(16 * k, 16)] = plsc.load_gather(
        ebuf, [ro, jnp.full((16,), 1, _i32)])
    cidx[pl.ds(16 * k, 16)] = plsc.load_gather(
        ebuf, [ro, jnp.full((16,), 2, _i32)])
    wv = plsc.load_gather(ebuf, [ro, jnp.full((16,), 3, _i32)])
    aval[pl.ds(16 + 16 * k, 16)] = jnp.abs(plsc.bitcast(wv, _f32))


@functools.partial(
    pl.kernel,
    out_type=[
        jax.ShapeDtypeStruct((NC, N, HID), _f32),
        jax.ShapeDtypeStruct((NC, N, 16), _f32),
    ],
    mesh=_MESH,
    compiler_params=_CP,
    scratch_types=[
        pltpu.VMEM((C, 4), _i32),
        pltpu.VMEM((C, 4), _i32),
        pltpu.VMEM((C,), _i32),
        pltpu.VMEM((C,), _i32),
        pltpu.VMEM((C,), _i32),
        pltpu.VMEM((C,), _i32),
        pltpu.VMEM((C + 16,), _f32),
        pltpu.VMEM((C + 16,), _f32),
        pltpu.VMEM((C, HID), _f32),
        pltpu.VMEM((C, HID), _f32),
        pltpu.VMEM((C, 16), _f32),
        pltpu.VMEM((40, 16), _f32),
        pltpu.VMEM_SHARED((N, HID), _f32),
        pltpu.VMEM_SHARED((N, 16), _f32),
        pltpu.SemaphoreType.DMA,
        pltpu.SemaphoreType.DMA,
        pltpu.SemaphoreType.DMA,
        pltpu.SemaphoreType.DMA,
        pltpu.SemaphoreType.DMA,
        pltpu.SemaphoreType.DMA,
    ],
)
def _sc_spmm128(ep_hbm, h_hbm, p_hbm, r_hbm,
                ebuf0, ebuf1, cidx0, cidx1, ridx0, ridx1, aval0, aval1,
                rows0, rows1, srs, zrs, acc, rsum,
                esem0, esem1, gsem0, gsem1, ssem, zsem):
  c = lax.axis_index("c")
  s = lax.axis_index("s")
  wid = s * NC + c
  zv = jnp.zeros((16,), _f32)
  e0 = jnp.where(lax.iota(_i32, 16) == 0, 1.0, 0.0).astype(_f32)

  def zr(i, carry):
    for j in range(HID // 16):
      rows0[i, pl.ds(j * 16, 16)] = zv
    return carry
  lax.fori_loop(0, C, zr, 0)

  def zr2(i, carry):
    zrs[i, :] = zv
    return carry
  lax.fori_loop(0, 40, zr2, 0)

  base = s * RPT
  zd = [pltpu.async_copy(rows0, acc.at[pl.ds(base + C * k, C), :], zsem)
        for k in range(4)]
  zd.append(pltpu.async_copy(rows0.at[pl.ds(0, 112), :],
                             acc.at[pl.ds(base + 512, 112), :], zsem))
  for d in zd:
    d.wait()

  @pl.when(s == NS - 1)
  def _():
    pltpu.async_copy(rows0.at[pl.ds(0, 16), :],
                     acc.at[pl.ds(NS * RPT, 16), :], zsem).wait()

  @pl.when(s < N // 1000)
  def _():
    zd2 = [pltpu.async_copy(zrs, rsum.at[pl.ds(s * 1000 + 40 * k, 40), :],
                            zsem) for k in range(25)]
    for d in zd2:
      d.wait()

  plsc.subcore_barrier()

  def scale(rows_p, aval_p):
    def sc(it, carry):
      for u in range(8):
        e = it * 8 + u
        sv = _splat(aval_p, e + 16)
        srs[e, :] = sv * e0
        for j in range(HID // 16):
          rows_p[e, pl.ds(j * 16, 16)] = rows_p[e, pl.ds(j * 16, 16)] * sv
      return carry
    lax.fori_loop(0, C // 8, sc, 0)

  ebase = wid * EPT

  def body(ii, carry):
    i0 = ii * 2
    b0 = pl.multiple_of(ebase + i0 * C, 8)
    b1 = pl.multiple_of(ebase + (i0 + 1) * C, 8)
    e_d0 = pltpu.async_copy(ep_hbm.at[pl.ds(b0, C), :], ebuf0, esem0)
    e_d1 = pltpu.async_copy(ep_hbm.at[pl.ds(b1, C), :], ebuf1, esem1)
    e_d0.wait()
    _deint(ebuf0, cidx0, ridx0, aval0)
    g0 = pltpu.async_copy(h_hbm.at[cidx0], rows0, gsem0)
    e_d1.wait()
    _deint(ebuf1, cidx1, ridx1, aval1)
    g1 = pltpu.async_copy(h_hbm.at[cidx1], rows1, gsem1)
    g0.wait()
    scale(rows0, aval0)
    pltpu.sync_copy(srs, rsum.at[ridx0], add=True)
    s0 = pltpu.async_copy(rows0, acc.at[ridx0], ssem, add=True)
    g1.wait()
    scale(rows1, aval1)
    pltpu.sync_copy(srs, rsum.at[ridx1], add=True)
    s0.wait()
    s1 = pltpu.async_copy(rows1, acc.at[ridx1], ssem, add=True)
    s1.wait()
    return carry

  lax.fori_loop(0, NCHUNK // 2, body, 0)
  plsc.subcore_barrier()

  pltpu.sync_copy(acc.at[pl.ds(base, RPT), :],
                  p_hbm.at[c, pl.ds(base, RPT), :])

  @pl.when(s == NS - 1)
  def _():
    pltpu.sync_copy(acc.at[pl.ds(NS * RPT, 16), :],
                    p_hbm.at[c, pl.ds(NS * RPT, 16), :])

  @pl.when(s < N // 1000)
  def _():
    pltpu.sync_copy(rsum.at[pl.ds(s * 1000, 1000), :],
                    r_hbm.at[c, pl.ds(s * 1000, 1000), :])


def _make_sc_spmm16(n_tables):
  nt = n_tables

  @functools.partial(
      pl.kernel,
      out_type=[jax.ShapeDtypeStruct((NC, N, OUT_C), _f32)
                for _ in range(nt)],
      mesh=_MESH,
      compiler_params=_CP,
      scratch_types=(
          [pltpu.VMEM((C, 4), _i32)] * 2
          + [pltpu.VMEM((C,), _i32)] * 4
          + [pltpu.VMEM((C + 16,), _f32)] * 2
          + [pltpu.VMEM((C, OUT_C), _f32) for _ in range(2 * nt)]
          + [pltpu.VMEM_SHARED((N, OUT_C), _f32) for _ in range(nt)]
          + [pltpu.SemaphoreType.DMA] * 6
      ),
  )
  def _sc_spmm16(*refs):
    tabs = refs[:nt]
    ep_hbm = refs[nt]
    outs = refs[nt + 1:2 * nt + 1]
    k = 2 * nt + 1
    ebuf0, ebuf1 = refs[k:k + 2]
    cidx0, cidx1, ridx0, ridx1 = refs[k + 2:k + 6]
    aval0, aval1 = refs[k + 6:k + 8]
    rows = [refs[k + 8 + 2 * t:k + 10 + 2 * t] for t in range(nt)]
    k2 = k + 8 + 2 * nt
    accs = refs[k2:k2 + nt]
    esem0, esem1, gsem0, gsem1, ssem, zsem = refs[k2 + nt:k2 + nt + 6]

    c = lax.axis_index("c")
    s = lax.axis_index("s")
    wid = s * NC + c
    zv = jnp.zeros((16,), _f32)

    def zr(i, carry):
      rows[0][0][i, :] = zv
      return carry
    lax.fori_loop(0, C, zr, 0)

    base = s * RPT
    zd = []
    for t in range(nt):
      zd += [pltpu.async_copy(rows[0][0],
                              accs[t].at[pl.ds(base + C * kk, C), :], zsem)
             for kk in range(4)]
      zd.append(pltpu.async_copy(rows[0][0].at[pl.ds(0, 112), :],
                                 accs[t].at[pl.ds(base + 512, 112), :], zsem))
    for d in zd:
      d.wait()

    @pl.when(s == NS - 1)
    def _():
      for t in range(nt):
        pltpu.async_copy(rows[0][0].at[pl.ds(0, 16), :],
                         accs[t].at[pl.ds(NS * RPT, 16), :], zsem).wait()

    plsc.subcore_barrier()

    def scale(slot, aval_p):
      def sc(it, carry):
        for u in range(8):
          e = it * 8 + u
          sv = _splat(aval_p, e + 16)
          for t in range(nt):
            rows[t][slot][e, :] = rows[t][slot][e, :] * sv
        return carry
      lax.fori_loop(0, C // 8, sc, 0)

    ebase = wid * EPT

    def body(ii, carry):
      i0 = ii * 2
      b0 = pl.multiple_of(ebase + i0 * C, 8)
      b1 = pl.multiple_of(ebase + (i0 + 1) * C, 8)
      e_d0 = pltpu.async_copy(ep_hbm.at[pl.ds(b0, C), :], ebuf0, esem0)
      e_d1 = pltpu.async_copy(ep_hbm.at[pl.ds(b1, C), :], ebuf1, esem1)
      e_d0.wait()
      _deint(ebuf0, cidx0, ridx0, aval0)
      g0 = [pltpu.async_copy(tabs[t].at[cidx0], rows[t][0], gsem0)
            for t in range(nt)]
      e_d1.wait()
      _deint(ebuf1, cidx1, ridx1, aval1)
      g1 = [pltpu.async_copy(tabs[t].at[cidx1], rows[t][1], gsem1)
            for t in range(nt)]
      for g in g0:
        g.wait()
      scale(0, aval0)
      s0 = [pltpu.async_copy(rows[t][0], accs[t].at[ridx0], ssem, add=True)
            for t in range(nt)]
      for g in g1:
        g.wait()
      scale(1, aval1)
      for d in s0:
        d.wait()
      s1 = [pltpu.async_copy(rows[t][1], accs[t].at[ridx1], ssem, add=True)
            for t in range(nt)]
      for d in s1:
        d.wait()
      return carry

    lax.fori_loop(0, NCHUNK // 2, body, 0)
    plsc.subcore_barrier()

    for t in range(nt):
      pltpu.sync_copy(accs[t].at[pl.ds(base, RPT), :],
                      outs[t].at[c, pl.ds(base, RPT), :])

      @pl.when(s == NS - 1)
      def _():
        pltpu.sync_copy(accs[t].at[pl.ds(NS * RPT, 16), :],
                        outs[t].at[c, pl.ds(NS * RPT, 16), :])

  return _sc_spmm16


_sc_spmm16x1 = _make_sc_spmm16(1)
_sc_spmm16x2 = _make_sc_spmm16(2)


_BM = 1000


def _tc_mm0(x, w0, b0):
  def body(x_ref, w_ref, b_ref, o_ref):
    o_ref[...] = jnp.dot(x_ref[...], w_ref[...],
                         preferred_element_type=_f32) + b_ref[...]
  return pl.pallas_call(
      body,
      grid=(N // _BM,),
      in_specs=[
          pl.BlockSpec((_BM, IN_C), lambda i: (i, 0)),
          pl.BlockSpec((IN_C, HID), lambda i: (0, 0)),
          pl.BlockSpec((1, HID), lambda i: (0, 0)),
      ],
      out_specs=pl.BlockSpec((_BM, HID), lambda i: (i, 0)),
      out_shape=jax.ShapeDtypeStruct((N, HID), _f32),
  )(x, w0, b0.reshape(1, HID))


def _tc_combine1(p, r3, w1, b1):
  def body(p_ref, r_ref, w_ref, b_ref, h2_ref, invd_ref):
    rs = (r_ref[0] + r_ref[1])[:, :1]
    den = jnp.where(rs > 0, rs, 1.0)
    inv = 1.0 / den
    hh = (p_ref[0] + p_ref[1]) * inv
    hh = jnp.maximum(hh, 0.0)
    h2_ref[...] = jnp.dot(hh, w_ref[...],
                          preferred_element_type=_f32) + b_ref[...]
    invd_ref[...] = inv
  return pl.pallas_call(
      body,
      grid=(N // _BM,),
      in_specs=[
          pl.BlockSpec((NC, _BM, HID), lambda i: (0, i, 0)),
          pl.BlockSpec((NC, _BM, 16), lambda i: (0, i, 0)),
          pl.BlockSpec((HID, OUT_C), lambda i: (0, 0)),
          pl.BlockSpec((1, OUT_C), lambda i: (0, 0)),
      ],
      out_specs=[
          pl.BlockSpec((_BM, OUT_C), lambda i: (i, 0)),
          pl.BlockSpec((_BM, 1), lambda i: (i, 0)),
      ],
      out_shape=[
          jax.ShapeDtypeStruct((N, OUT_C), _f32),
          jax.ShapeDtypeStruct((N, 1), _f32),
      ],
  )(p, r3, w1, b1.reshape(1, OUT_C))


def _tc_combine16(p, invd):
  def body(p_ref, i_ref, o_ref):
    o_ref[...] = (p_ref[0] + p_ref[1]) * i_ref[...]
  return pl.pallas_call(
      body,
      grid=(N // _BM,),
      in_specs=[
          pl.BlockSpec((NC, _BM, OUT_C), lambda i: (0, i, 0)),
          pl.BlockSpec((_BM, 1), lambda i: (i, 0)),
      ],
      out_specs=pl.BlockSpec((_BM, OUT_C), lambda i: (i, 0)),
      out_shape=jax.ShapeDtypeStruct((N, OUT_C), _f32),
  )(p, invd)


def kernel(x, soft_labels, edge_weights, W0, b0, W1, b1, edge_index):
  row = edge_index[0]
  col = edge_index[1]
  wbits = lax.bitcast_convert_type(edge_weights, _i32)
  padi = jnp.zeros((E_PAD - E,), _i32)
  rowp = jnp.concatenate([row, padi])
  colp = jnp.concatenate([col, padi])
  wp = jnp.concatenate([wbits, padi])
  epack = jnp.stack([jnp.zeros_like(rowp), rowp, colp, wp], axis=1)

  h = _tc_mm0(x, W0, b0)
  p, r = _sc_spmm128(epack, h)
  h2, invd = _tc_combine1(p, r, W1, b1)

  p_out, p_l = _sc_spmm16x2(h2, soft_labels, epack)
  out = _tc_combine16(p_out, invd)
  l1 = _tc_combine16(p_l, invd)

  (p_l2,) = _sc_spmm16x1(l1, epack)
  l2 = _tc_combine16(p_l2, invd)
  (p_l3,) = _sc_spmm16x1(l2, epack)
  labels = _tc_combine16(p_l3, invd)

  return (out, labels)


# speedup vs baseline: 1.2474x; 1.2474x over previous
import functools

import jax
import jax.numpy as jnp
from jax import lax
from jax.experimental import pallas as pl
from jax.experimental.pallas import tpu as pltpu
from jax.experimental.pallas import tpu_sc as plsc

N = 10000
E = 320000
IN_C = 128
HID = 128
OUT_C = 16

NC = 2
NS = 16
NW = NC * NS
C = 128
EPT = 10240
E_PAD = EPT * NW
NCHUNK = EPT // C
RPT = 624

_f32 = jnp.float32
_i32 = jnp.int32

_MESH = plsc.VectorSubcoreMesh(
    core_axis_name="c", subcore_axis_name="s", num_cores=NC, num_subcores=NS)

_CP = pltpu.CompilerParams(needs_layout_passes=False, use_tc_tiling_on_sc=False)


def _splat(vec_ref, e):
  return plsc.load_gather(vec_ref, [jnp.full((16,), e, _i32)])


def _absv(aval):
  for k in range(C // 16):
    aval[pl.ds(16 + k * 16, 16)] = jnp.abs(aval[pl.ds(16 + k * 16, 16)])


@functools.partial(
    pl.kernel,
    out_type=[
        jax.ShapeDtypeStruct((NC, N, HID), _f32),
        jax.ShapeDtypeStruct((NC, N, 16), _f32),
    ],
    mesh=_MESH,
    compiler_params=_CP,
    scratch_types=[
        pltpu.VMEM((C,), _i32),
        pltpu.VMEM((C,), _i32),
        pltpu.VMEM((C,), _i32),
        pltpu.VMEM((C,), _i32),
        pltpu.VMEM((C + 16,), _f32),
        pltpu.VMEM((C + 16,), _f32),
        pltpu.VMEM((C, HID), _f32),
        pltpu.VMEM((C, HID), _f32),
        pltpu.VMEM((C, 16), _f32),
        pltpu.VMEM((40, 16), _f32),
        pltpu.VMEM_SHARED((N, HID), _f32),
        pltpu.VMEM_SHARED((N, 16), _f32),
        pltpu.SemaphoreType.DMA,
        pltpu.SemaphoreType.DMA,
        pltpu.SemaphoreType.DMA,
        pltpu.SemaphoreType.DMA,
        pltpu.SemaphoreType.DMA,
        pltpu.SemaphoreType.DMA,
    ],
)
def _sc_spmm128(row_hbm, col_hbm, w_hbm, h_hbm, p_hbm, r_hbm,
                cidx0, cidx1, ridx0, ridx1, aval0, aval1,
                rows0, rows1, srs, zrs, acc, rsum,
                esem0, esem1, gsem0, gsem1, ssem, zsem):
  c = lax.axis_index("c")
  s = lax.axis_index("s")
  wid = s * NC + c
  zv = jnp.zeros((16,), _f32)
  e0 = jnp.where(lax.iota(_i32, 16) == 0, 1.0, 0.0).astype(_f32)

  def zr(i, carry):
    for j in range(HID // 16):
      rows0[i, pl.ds(j * 16, 16)] = zv
    return carry
  lax.fori_loop(0, C, zr, 0)

  def zr2(i, carry):
    zrs[i, :] = zv
    return carry
  lax.fori_loop(0, 40, zr2, 0)

  base = s * RPT
  zd = [pltpu.async_copy(rows0, acc.at[pl.ds(base + C * k, C), :], zsem)
        for k in range(4)]
  zd.append(pltpu.async_copy(rows0.at[pl.ds(0, 112), :],
                             acc.at[pl.ds(base + 512, 112), :], zsem))
  for d in zd:
    d.wait()

  @pl.when(s == NS - 1)
  def _():
    pltpu.async_copy(rows0.at[pl.ds(0, 16), :],
                     acc.at[pl.ds(NS * RPT, 16), :], zsem).wait()

  @pl.when(s < N // 1000)
  def _():
    zd2 = [pltpu.async_copy(zrs, rsum.at[pl.ds(s * 1000 + 40 * k, 40), :],
                            zsem) for k in range(25)]
    for d in zd2:
      d.wait()

  plsc.subcore_barrier()

  def scale(rows_p, aval_p):
    def sc(it, carry):
      for u in range(8):
        e = it * 8 + u
        sv = _splat(aval_p, e + 16)
        srs[e, :] = sv * e0
        for j in range(HID // 16):
          rows_p[e, pl.ds(j * 16, 16)] = rows_p[e, pl.ds(j * 16, 16)] * sv
      return carry
    lax.fori_loop(0, C // 8, sc, 0)

  ebase = wid * EPT

  def body(ii, carry):
    i0 = ii * 2
    b0 = pl.multiple_of(ebase + i0 * C, 8)
    b1 = pl.multiple_of(ebase + (i0 + 1) * C, 8)
    e_d0 = [pltpu.async_copy(col_hbm.at[pl.ds(b0, C)], cidx0, esem0),
            pltpu.async_copy(row_hbm.at[pl.ds(b0, C)], ridx0, esem0),
            pltpu.async_copy(w_hbm.at[pl.ds(b0, C)], aval0.at[pl.ds(16, C)],
                             esem0)]
    e_d1 = [pltpu.async_copy(col_hbm.at[pl.ds(b1, C)], cidx1, esem1),
            pltpu.async_copy(row_hbm.at[pl.ds(b1, C)], ridx1, esem1),
            pltpu.async_copy(w_hbm.at[pl.ds(b1, C)], aval1.at[pl.ds(16, C)],
                             esem1)]
    for d in e_d0:
      d.wait()
    _absv(aval0)
    g0 = pltpu.async_copy(h_hbm.at[cidx0], rows0, gsem0)
    for d in e_d1:
      d.wait()
    _absv(aval1)
    g1 = pltpu.async_copy(h_hbm.at[cidx1], rows1, gsem1)
    g0.wait()
    scale(rows0, aval0)
    pltpu.sync_copy(srs, rsum.at[ridx0], add=True)
    s0 = pltpu.async_copy(rows0, acc.at[ridx0], ssem, add=True)
    g1.wait()
    scale(rows1, aval1)
    pltpu.sync_copy(srs, rsum.at[ridx1], add=True)
    s0.wait()
    s1 = pltpu.async_copy(rows1, acc.at[ridx1], ssem, add=True)
    s1.wait()
    return carry

  lax.fori_loop(0, NCHUNK // 2, body, 0)
  plsc.subcore_barrier()

  pltpu.sync_copy(acc.at[pl.ds(base, RPT), :],
                  p_hbm.at[c, pl.ds(base, RPT), :])

  @pl.when(s == NS - 1)
  def _():
    pltpu.sync_copy(acc.at[pl.ds(NS * RPT, 16), :],
                    p_hbm.at[c, pl.ds(NS * RPT, 16), :])

  @pl.when(s < N // 1000)
  def _():
    pltpu.sync_copy(rsum.at[pl.ds(s * 1000, 1000), :],
                    r_hbm.at[c, pl.ds(s * 1000, 1000), :])


def _make_sc_spmm16(n_tables):
  nt = n_tables

  @functools.partial(
      pl.kernel,
      out_type=[jax.ShapeDtypeStruct((NC, N, OUT_C), _f32)
                for _ in range(nt)],
      mesh=_MESH,
      compiler_params=_CP,
      scratch_types=(
          [pltpu.VMEM((C,), _i32)] * 4
          + [pltpu.VMEM((C + 16,), _f32)] * 2
          + [pltpu.VMEM((C, OUT_C), _f32) for _ in range(2 * nt)]
          + [pltpu.VMEM_SHARED((N, OUT_C), _f32) for _ in range(nt)]
          + [pltpu.SemaphoreType.DMA] * 6
      ),
  )
  def _sc_spmm16(*refs):
    tabs = refs[:nt]
    row_hbm, col_hbm, w_hbm = refs[nt:nt + 3]
    outs = refs[nt + 3:2 * nt + 3]
    k = 2 * nt + 3
    cidx0, cidx1, ridx0, ridx1 = refs[k:k + 4]
    aval0, aval1 = refs[k + 4:k + 6]
    rows = [refs[k + 6 + 2 * t:k + 8 + 2 * t] for t in range(nt)]
    k2 = k + 6 + 2 * nt
    accs = refs[k2:k2 + nt]
    esem0, esem1, gsem0, gsem1, ssem, zsem = refs[k2 + nt:k2 + nt + 6]

    c = lax.axis_index("c")
    s = lax.axis_index("s")
    wid = s * NC + c
    zv = jnp.zeros((16,), _f32)

    def zr(i, carry):
      rows[0][0][i, :] = zv
      return carry
    lax.fori_loop(0, C, zr, 0)

    base = s * RPT
    zd = []
    for t in range(nt):
      zd += [pltpu.async_copy(rows[0][0],
                              accs[t].at[pl.ds(base + C * kk, C), :], zsem)
             for kk in range(4)]
      zd.append(pltpu.async_copy(rows[0][0].at[pl.ds(0, 112), :],
                                 accs[t].at[pl.ds(base + 512, 112), :], zsem))
    for d in zd:
      d.wait()

    @pl.when(s == NS - 1)
    def _():
      for t in range(nt):
        pltpu.async_copy(rows[0][0].at[pl.ds(0, 16), :],
                         accs[t].at[pl.ds(NS * RPT, 16), :], zsem).wait()

    plsc.subcore_barrier()

    def scale(slot, aval_p):
      def sc(it, carry):
        for u in range(8):
          e = it * 8 + u
          sv = _splat(aval_p, e + 16)
          for t in range(nt):
            rows[t][slot][e, :] = rows[t][slot][e, :] * sv
        return carry
      lax.fori_loop(0, C // 8, sc, 0)

    ebase = wid * EPT

    def body(ii, carry):
      i0 = ii * 2
      b0 = pl.multiple_of(ebase + i0 * C, 8)
      b1 = pl.multiple_of(ebase + (i0 + 1) * C, 8)
      e_d0 = [pltpu.async_copy(col_hbm.at[pl.ds(b0, C)], cidx0, esem0),
              pltpu.async_copy(row_hbm.at[pl.ds(b0, C)], ridx0, esem0),
              pltpu.async_copy(w_hbm.at[pl.ds(b0, C)], aval0.at[pl.ds(16, C)],
                               esem0)]
      e_d1 = [pltpu.async_copy(col_hbm.at[pl.ds(b1, C)], cidx1, esem1),
              pltpu.async_copy(row_hbm.at[pl.ds(b1, C)], ridx1, esem1),
              pltpu.async_copy(w_hbm.at[pl.ds(b1, C)], aval1.at[pl.ds(16, C)],
                               esem1)]
      for d in e_d0:
        d.wait()
      _absv(aval0)
      g0 = [pltpu.async_copy(tabs[t].at[cidx0], rows[t][0], gsem0)
            for t in range(nt)]
      for d in e_d1:
        d.wait()
      _absv(aval1)
      g1 = [pltpu.async_copy(tabs[t].at[cidx1], rows[t][1], gsem1)
            for t in range(nt)]
      for g in g0:
        g.wait()
      scale(0, aval0)
      s0 = [pltpu.async_copy(rows[t][0], accs[t].at[ridx0], ssem, add=True)
            for t in range(nt)]
      for g in g1:
        g.wait()
      scale(1, aval1)
      for d in s0:
        d.wait()
      s1 = [pltpu.async_copy(rows[t][1], accs[t].at[ridx1], ssem, add=True)
            for t in range(nt)]
      for d in s1:
        d.wait()
      return carry

    lax.fori_loop(0, NCHUNK // 2, body, 0)
    plsc.subcore_barrier()

    for t in range(nt):
      pltpu.sync_copy(accs[t].at[pl.ds(base, RPT), :],
                      outs[t].at[c, pl.ds(base, RPT), :])

      @pl.when(s == NS - 1)
      def _():
        pltpu.sync_copy(accs[t].at[pl.ds(NS * RPT, 16), :],
                        outs[t].at[c, pl.ds(NS * RPT, 16), :])

  return _sc_spmm16


_sc_spmm16x1 = _make_sc_spmm16(1)
_sc_spmm16x2 = _make_sc_spmm16(2)


_BM = 1000


def _tc_mm0(x, w0, b0):
  def body(x_ref, w_ref, b_ref, o_ref):
    o_ref[...] = jnp.dot(x_ref[...], w_ref[...],
                         preferred_element_type=_f32) + b_ref[...]
  return pl.pallas_call(
      body,
      grid=(N // _BM,),
      in_specs=[
          pl.BlockSpec((_BM, IN_C), lambda i: (i, 0)),
          pl.BlockSpec((IN_C, HID), lambda i: (0, 0)),
          pl.BlockSpec((1, HID), lambda i: (0, 0)),
      ],
      out_specs=pl.BlockSpec((_BM, HID), lambda i: (i, 0)),
      out_shape=jax.ShapeDtypeStruct((N, HID), _f32),
  )(x, w0, b0.reshape(1, HID))


def _tc_combine1(p, r3, w1, b1):
  def body(p_ref, r_ref, w_ref, b_ref, h2_ref, invd_ref):
    rs = (r_ref[0] + r_ref[1])[:, :1]
    den = jnp.where(rs > 0, rs, 1.0)
    inv = 1.0 / den
    hh = (p_ref[0] + p_ref[1]) * inv
    hh = jnp.maximum(hh, 0.0)
    h2_ref[...] = jnp.dot(hh, w_ref[...],
                          preferred_element_type=_f32) + b_ref[...]
    invd_ref[...] = inv
  return pl.pallas_call(
      body,
      grid=(N // _BM,),
      in_specs=[
          pl.BlockSpec((NC, _BM, HID), lambda i: (0, i, 0)),
          pl.BlockSpec((NC, _BM, 16), lambda i: (0, i, 0)),
          pl.BlockSpec((HID, OUT_C), lambda i: (0, 0)),
          pl.BlockSpec((1, OUT_C), lambda i: (0, 0)),
      ],
      out_specs=[
          pl.BlockSpec((_BM, OUT_C), lambda i: (i, 0)),
          pl.BlockSpec((_BM, 1), lambda i: (i, 0)),
      ],
      out_shape=[
          jax.ShapeDtypeStruct((N, OUT_C), _f32),
          jax.ShapeDtypeStruct((N, 1), _f32),
      ],
  )(p, r3, w1, b1.reshape(1, OUT_C))


def _tc_combine16(p, invd):
  def body(p_ref, i_ref, o_ref):
    o_ref[...] = (p_ref[0] + p_ref[1]) * i_ref[...]
  return pl.pallas_call(
      body,
      grid=(N // _BM,),
      in_specs=[
          pl.BlockSpec((NC, _BM, OUT_C), lambda i: (0, i, 0)),
          pl.BlockSpec((_BM, 1), lambda i: (i, 0)),
      ],
      out_specs=pl.BlockSpec((_BM, OUT_C), lambda i: (i, 0)),
      out_shape=jax.ShapeDtypeStruct((N, OUT_C), _f32),
  )(p, invd)


def kernel(x, soft_labels, edge_weights, W0, b0, W1, b1, edge_index):
  padi = jnp.zeros((E_PAD - E,), _i32)
  padf = jnp.zeros((E_PAD - E,), _f32)
  rowp = jnp.concatenate([edge_index[0], padi])
  colp = jnp.concatenate([edge_index[1], padi])
  wp = jnp.concatenate([edge_weights, padf])

  h = _tc_mm0(x, W0, b0)
  p, r = _sc_spmm128(rowp, colp, wp, h)
  h2, invd = _tc_combine1(p, r, W1, b1)

  p_out, p_l = _sc_spmm16x2(h2, soft_labels, rowp, colp, wp)
  out = _tc_combine16(p_out, invd)
  l1 = _tc_combine16(p_l, invd)

  (p_l2,) = _sc_spmm16x1(l1, rowp, colp, wp)
  l2 = _tc_combine16(p_l2, invd)
  (p_l3,) = _sc_spmm16x1(l2, rowp, colp, wp)
  labels = _tc_combine16(p_l3, invd)

  return (out, labels)
